# baseline (speedup 1.0000x reference)
import functools

import jax
import jax.numpy as jnp
from jax import lax
from jax.experimental import pallas as pl
from jax.experimental.pallas import tpu as pltpu
from jax.experimental.pallas import tpu_sc as plsc

_ROWS = 16384
_COLS = 256
_NC, _NS = 2, 16
_NW = _NC * _NS
_W_ROWS = _ROWS // _NW
_CH_ROWS = 128
_N_CH = _W_ROWS // _CH_ROWS


def _sc_copy_body(t_hbm, q_hbm, t_out, q_out, buf_a, buf_b, sems):
    wid = lax.axis_index("s") * _NC + lax.axis_index("c")
    base = wid * _W_ROWS

    jobs = []
    for src, dst in ((t_hbm, t_out), (q_hbm, q_out)):
        for c in range(_N_CH):
            sl = pl.ds(base + c * _CH_ROWS, _CH_ROWS)
            jobs.append((src.at[sl], dst.at[sl]))

    bufs = (buf_a, buf_b)
    n = len(jobs)
    loads = [None] * n
    stores = [None] * n
    loads[0] = pltpu.async_copy(jobs[0][0], bufs[0], sems.at[0])
    for j in range(n):
        b = j % 2
        loads[j].wait()
        stores[j] = pltpu.async_copy(bufs[b], jobs[j][1], sems.at[2 + b])
        if j + 1 < n:
            if j - 1 >= 0:
                stores[j - 1].wait()
            loads[j + 1] = pltpu.async_copy(
                jobs[j + 1][0], bufs[(j + 1) % 2], sems.at[(j + 1) % 2]
            )
    stores[n - 2].wait()
    stores[n - 1].wait()


def kernel(emb_targets, emb_queries):
    mesh = plsc.VectorSubcoreMesh(
        core_axis_name="c", subcore_axis_name="s", num_cores=_NC, num_subcores=_NS
    )
    sc_copy = functools.partial(
        pl.kernel,
        mesh=mesh,
        out_type=[
            jax.ShapeDtypeStruct((_ROWS, _COLS), jnp.float32),
            jax.ShapeDtypeStruct((_ROWS, _COLS), jnp.float32),
        ],
        scratch_types=[
            pltpu.VMEM((_CH_ROWS, _COLS), jnp.float32),
            pltpu.VMEM((_CH_ROWS, _COLS), jnp.float32),
            pltpu.SemaphoreType.DMA((4,)),
        ],
    )(_sc_copy_body)
    return tuple(sc_copy(emb_targets, emb_queries))


# speedup vs baseline: 1.0112x; 1.0112x over previous
import functools

import jax
import jax.numpy as jnp
from jax import lax
from jax.experimental import pallas as pl
from jax.experimental.pallas import tpu as pltpu
from jax.experimental.pallas import tpu_sc as plsc

_ROWS = 16384
_COLS = 256
_NC, _NS = 2, 16
_NW = _NC * _NS
_W_ROWS = _ROWS // _NW
_CH_ROWS = 32
_N_CH = _W_ROWS // _CH_ROWS
_RINGS = 4


def _sc_copy_body(t_hbm, q_hbm, t_out, q_out, *scratch):
    bufs = scratch[: 2 * _RINGS]
    sems = scratch[2 * _RINGS]
    wid = lax.axis_index("s") * _NC + lax.axis_index("c")
    base = wid * _W_ROWS

    jobs = []
    for src, dst in ((t_hbm, t_out), (q_hbm, q_out)):
        for c in range(_N_CH):
            sl = pl.ds(base + c * _CH_ROWS, _CH_ROWS)
            jobs.append((src.at[sl], dst.at[sl]))

    n = len(jobs)
    steps = n // _RINGS
    loads = [None] * n
    stores = [None] * n

    def buf(r, p):
        return bufs[2 * r + p % 2]

    def lsem(r):
        return sems.at[r]

    def ssem(r, p):
        return sems.at[_RINGS + 2 * r + p % 2]

    for r in range(_RINGS):
        loads[r] = pltpu.async_copy(jobs[r][0], buf(r, 0), lsem(r))
    for p in range(steps):
        for r in range(_RINGS):
            j = p * _RINGS + r
            loads[j].wait()
            stores[j] = pltpu.async_copy(buf(r, p), jobs[j][1], ssem(r, p))
            if p + 1 < steps:
                if p - 1 >= 0:
                    stores[j - _RINGS].wait()
                loads[j + _RINGS] = pltpu.async_copy(
                    jobs[j + _RINGS][0], buf(r, p + 1), lsem(r)
                )
    for r in range(_RINGS):
        stores[(steps - 2) * _RINGS + r].wait()
        stores[(steps - 1) * _RINGS + r].wait()


def kernel(emb_targets, emb_queries):
    mesh = plsc.VectorSubcoreMesh(
        core_axis_name="c", subcore_axis_name="s", num_cores=_NC, num_subcores=_NS
    )
    sc_copy = functools.partial(
        pl.kernel,
        mesh=mesh,
        out_type=[
            jax.ShapeDtypeStruct((_ROWS, _COLS), jnp.float32),
            jax.ShapeDtypeStruct((_ROWS, _COLS), jnp.float32),
        ],
        scratch_types=(
            [pltpu.VMEM((_CH_ROWS, _COLS), jnp.float32) for _ in range(2 * _RINGS)]
            + [pltpu.SemaphoreType.DMA((3 * _RINGS,))]
        ),
    )(_sc_copy_body)
    return tuple(sc_copy(emb_targets, emb_queries))


# speedup vs baseline: 1.6233x; 1.6053x over previous
import jax
import jax.numpy as jnp
from jax.experimental import pallas as pl
from jax.experimental.pallas import tpu as pltpu

_ROWS = 16384
_COLS = 256
_CH_ROWS = 512
_RINGS = 4
_N_CH = _ROWS // _CH_ROWS


def _dma_ring_body(t_hbm, q_hbm, t_out, q_out, *scratch):
    bufs = scratch[: 2 * _RINGS]
    sems = scratch[2 * _RINGS]

    jobs = []
    for src, dst in ((t_hbm, t_out), (q_hbm, q_out)):
        for c in range(_N_CH):
            sl = pl.ds(c * _CH_ROWS, _CH_ROWS)
            jobs.append((src.at[sl], dst.at[sl]))

    n = len(jobs)
    steps = n // _RINGS
    loads = [None] * n
    stores = [None] * n

    def buf(r, p):
        return bufs[2 * r + p % 2]

    def lsem(r):
        return sems.at[r]

    def ssem(r, p):
        return sems.at[_RINGS + 2 * r + p % 2]

    for r in range(_RINGS):
        loads[r] = pltpu.make_async_copy(jobs[r][0], buf(r, 0), lsem(r))
        loads[r].start()
    for p in range(steps):
        for r in range(_RINGS):
            j = p * _RINGS + r
            loads[j].wait()
            stores[j] = pltpu.make_async_copy(buf(r, p), jobs[j][1], ssem(r, p))
            stores[j].start()
            if p + 1 < steps:
                if p - 1 >= 0:
                    stores[j - _RINGS].wait()
                loads[j + _RINGS] = pltpu.make_async_copy(
                    jobs[j + _RINGS][0], buf(r, p + 1), lsem(r)
                )
                loads[j + _RINGS].start()
    for r in range(_RINGS):
        stores[(steps - 2) * _RINGS + r].wait()
        stores[(steps - 1) * _RINGS + r].wait()


def kernel(emb_targets, emb_queries):
    any_spec = pl.BlockSpec(memory_space=pl.MemorySpace.ANY)
    out_t, out_q = pl.pallas_call(
        _dma_ring_body,
        in_specs=[any_spec, any_spec],
        out_specs=[any_spec, any_spec],
        out_shape=[
            jax.ShapeDtypeStruct((_ROWS, _COLS), jnp.float32),
            jax.ShapeDtypeStruct((_ROWS, _COLS), jnp.float32),
        ],
        scratch_shapes=(
            [pltpu.VMEM((_CH_ROWS, _COLS), jnp.float32) for _ in range(2 * _RINGS)]
            + [pltpu.SemaphoreType.DMA((3 * _RINGS,))]
        ),
    )(emb_targets, emb_queries)
    return (out_t, out_q)


# speedup vs baseline: 2.0669x; 1.2733x over previous
import jax
import jax.numpy as jnp
from jax.experimental import pallas as pl

_ROWS = 16384
_COLS = 256
_BLOCK_ROWS = 4096


def _copy_body(t_ref, q_ref, t_out, q_out):
    t_out[...] = t_ref[...]
    q_out[...] = q_ref[...]


def kernel(emb_targets, emb_queries):
    grid = (_ROWS // _BLOCK_ROWS,)
    spec = pl.BlockSpec((_BLOCK_ROWS, _COLS), lambda i: (i, 0))
    out_t, out_q = pl.pallas_call(
        _copy_body,
        grid=grid,
        in_specs=[spec, spec],
        out_specs=[spec, spec],
        out_shape=[
            jax.ShapeDtypeStruct((_ROWS, _COLS), jnp.float32),
            jax.ShapeDtypeStruct((_ROWS, _COLS), jnp.float32),
        ],
    )(emb_targets, emb_queries)
    return (out_t, out_q)


# speedup vs baseline: 2.2043x; 1.0665x over previous
import jax
import jax.numpy as jnp
from jax.experimental import pallas as pl

_ROWS = 16384
_COLS = 256
_BLOCK_ROWS = 6144


def _copy_body(t_ref, q_ref, t_out, q_out):
    t_out[...] = t_ref[...]
    q_out[...] = q_ref[...]


def kernel(emb_targets, emb_queries):
    grid = (-(-_ROWS // _BLOCK_ROWS),)
    spec = pl.BlockSpec((_BLOCK_ROWS, _COLS), lambda i: (i, 0))
    out_t, out_q = pl.pallas_call(
        _copy_body,
        grid=grid,
        in_specs=[spec, spec],
        out_specs=[spec, spec],
        out_shape=[
            jax.ShapeDtypeStruct((_ROWS, _COLS), jnp.float32),
            jax.ShapeDtypeStruct((_ROWS, _COLS), jnp.float32),
        ],
    )(emb_targets, emb_queries)
    return (out_t, out_q)
